# baseline (speedup 1.0000x reference)
import functools

import jax
import jax.numpy as jnp
from jax import lax
from jax.experimental import pallas as pl
from jax.experimental.pallas import tpu as pltpu
from jax.experimental.pallas import tpu_sc as plsc

DIM = 64
BATCH = 16384

_INFO = plsc.get_sparse_core_info()
_NC = _INFO.num_cores
_NS = _INFO.num_subcores
_NW = _NC * _NS
_PER_W = BATCH // _NW
_GCHUNK = 128
_NG = _PER_W // _GCHUNK
_EPS = 1e-12


def _gather_body(hidx_hbm, ridx_hbm, tidx_hbm, ent_hbm, rel_hbm,
                 hrows_hbm, rrows_hbm, trows_hbm,
                 idxh_v, idxr_v, idxt_v, bufh_v, bufr_v, buft_v,
                 semh, semr, semt):
    wid = lax.axis_index("s") * _NC + lax.axis_index("c")

    pltpu.sync_copy(hidx_hbm.at[pl.ds(wid * _NG, _NG)], idxh_v)
    pltpu.sync_copy(ridx_hbm.at[pl.ds(wid * _NG, _NG)], idxr_v)
    pltpu.sync_copy(tidx_hbm.at[pl.ds(wid * _NG, _NG)], idxt_v)

    for j in range(_NG):
        ch = pltpu.async_copy(ent_hbm.at[idxh_v.at[j]], bufh_v, semh)
        ct = pltpu.async_copy(ent_hbm.at[idxt_v.at[j]], buft_v, semt)
        cr = pltpu.async_copy(rel_hbm.at[idxr_v.at[j]], bufr_v, semr)
        dst = pl.ds(wid * _PER_W + j * _GCHUNK, _GCHUNK)
        ch.wait()
        pltpu.sync_copy(bufh_v, hrows_hbm.at[dst])
        ct.wait()
        pltpu.sync_copy(buft_v, trows_hbm.at[dst])
        cr.wait()
        pltpu.sync_copy(bufr_v, rrows_hbm.at[dst])


def _sc_gather(hidx, ridx, tidx, ent_emb, rel_emb):
    mesh = plsc.VectorSubcoreMesh(core_axis_name="c", subcore_axis_name="s")
    rows_t = jax.ShapeDtypeStruct((BATCH, DIM), jnp.float32)
    k = functools.partial(
        pl.kernel,
        mesh=mesh,
        compiler_params=pltpu.CompilerParams(use_tc_tiling_on_sc=False),
        out_type=[rows_t, rows_t, rows_t],
        scratch_types=[
            pltpu.VMEM((_NG, _GCHUNK), jnp.int32),
            pltpu.VMEM((_NG, _GCHUNK), jnp.int32),
            pltpu.VMEM((_NG, _GCHUNK), jnp.int32),
            pltpu.VMEM((_GCHUNK, DIM), jnp.float32),
            pltpu.VMEM((_GCHUNK, DIM), jnp.float32),
            pltpu.VMEM((_GCHUNK, DIM), jnp.float32),
            pltpu.SemaphoreType.DMA,
            pltpu.SemaphoreType.DMA,
            pltpu.SemaphoreType.DMA,
        ],
    )(_gather_body)
    return k(hidx, ridx, tidx, ent_emb, rel_emb)


def _score_body(h_ref, r_ref, t_ref, o_ref):
    h = h_ref[...]
    r = r_ref[...]
    t = t_ref[...]
    sh = jnp.maximum(jnp.sum(jnp.abs(h), axis=1, keepdims=True), _EPS)
    st = jnp.maximum(jnp.sum(jnp.abs(t), axis=1, keepdims=True), _EPS)
    d = jnp.abs(h / sh + r - t / st)
    o_ref[...] = -jnp.sum(d, axis=1)


def _tc_score(hrows, rrows, trows):
    blk = 1024
    grid = BATCH // blk
    spec = pl.BlockSpec((blk, DIM), lambda i: (i, 0))
    return pl.pallas_call(
        _score_body,
        grid=(grid,),
        in_specs=[spec, spec, spec],
        out_specs=pl.BlockSpec((blk,), lambda i: (i,)),
        out_shape=jax.ShapeDtypeStruct((BATCH,), jnp.float32),
    )(hrows, rrows, trows)


def kernel(head_idxs, rel_idxs, tail_idxs, ent_emb, rel_emb):
    hidx = head_idxs.astype(jnp.int32).reshape(BATCH // _GCHUNK, _GCHUNK)
    ridx = rel_idxs.astype(jnp.int32).reshape(BATCH // _GCHUNK, _GCHUNK)
    tidx = tail_idxs.astype(jnp.int32).reshape(BATCH // _GCHUNK, _GCHUNK)
    hrows, rrows, trows = _sc_gather(hidx, ridx, tidx, ent_emb, rel_emb)
    return _tc_score(hrows, rrows, trows)


# speedup vs baseline: 1.6778x; 1.6778x over previous
import functools

import jax
import jax.numpy as jnp
from jax import lax
from jax.experimental import pallas as pl
from jax.experimental.pallas import tpu as pltpu
from jax.experimental.pallas import tpu_sc as plsc

DIM = 64
BATCH = 16384

_INFO = plsc.get_sparse_core_info()
_NC = _INFO.num_cores
_NS = _INFO.num_subcores
_NW = _NC * _NS
_PER_W = BATCH // _NW
_CHUNK = 256
_NCH = _PER_W // _CHUNK
_EPS = 1e-12


def _gather_body(hidx_hbm, ridx_hbm, tidx_hbm, ent_hbm, rel_hbm,
                 hrows_hbm, rrows_hbm, trows_hbm,
                 idxh_v, idxr_v, idxt_v, bufh_v, bufr_v, buft_v,
                 semh, semr, semt):
    wid = lax.axis_index("s") * _NC + lax.axis_index("c")
    base = wid * _PER_W

    pltpu.sync_copy(hidx_hbm.at[pl.ds(base, _PER_W)], idxh_v)
    pltpu.sync_copy(ridx_hbm.at[pl.ds(base, _PER_W)], idxr_v)
    pltpu.sync_copy(tidx_hbm.at[pl.ds(base, _PER_W)], idxt_v)

    def make_fire(idx_v, tab_hbm, buf_v, sem, off):
        def fire(g, carry):
            v = idx_v[pl.ds(off + g * 16, 16)]
            for i in range(16):
                pltpu.async_copy(tab_hbm.at[pl.ds(v[i], 1)],
                                 buf_v.at[pl.ds(g * 16 + i, 1)], sem)
            return carry
        return fire

    for j in range(_NCH):
        off = j * _CHUNK
        lax.fori_loop(0, _CHUNK // 16, make_fire(idxh_v, ent_hbm, bufh_v, semh, off), 0)
        lax.fori_loop(0, _CHUNK // 16, make_fire(idxt_v, ent_hbm, buft_v, semt, off), 0)
        lax.fori_loop(0, _CHUNK // 16, make_fire(idxr_v, rel_hbm, bufr_v, semr, off), 0)
        dst = pl.ds(base + off, _CHUNK)
        pltpu.make_async_copy(ent_hbm.at[pl.ds(0, _CHUNK)], bufh_v, semh).wait()
        pltpu.sync_copy(bufh_v, hrows_hbm.at[dst])
        pltpu.make_async_copy(ent_hbm.at[pl.ds(0, _CHUNK)], buft_v, semt).wait()
        pltpu.sync_copy(buft_v, trows_hbm.at[dst])
        pltpu.make_async_copy(ent_hbm.at[pl.ds(0, _CHUNK)], bufr_v, semr).wait()
        pltpu.sync_copy(bufr_v, rrows_hbm.at[dst])


def _sc_gather(hidx, ridx, tidx, ent_emb, rel_emb):
    mesh = plsc.VectorSubcoreMesh(core_axis_name="c", subcore_axis_name="s")
    rows_t = jax.ShapeDtypeStruct((BATCH, DIM), jnp.float32)
    k = functools.partial(
        pl.kernel,
        mesh=mesh,
        out_type=[rows_t, rows_t, rows_t],
        scratch_types=[
            pltpu.VMEM((_PER_W,), jnp.int32),
            pltpu.VMEM((_PER_W,), jnp.int32),
            pltpu.VMEM((_PER_W,), jnp.int32),
            pltpu.VMEM((_CHUNK, DIM), jnp.float32),
            pltpu.VMEM((_CHUNK, DIM), jnp.float32),
            pltpu.VMEM((_CHUNK, DIM), jnp.float32),
            pltpu.SemaphoreType.DMA,
            pltpu.SemaphoreType.DMA,
            pltpu.SemaphoreType.DMA,
        ],
    )(_gather_body)
    return k(hidx, ridx, tidx, ent_emb, rel_emb)


def _score_body(h_ref, r_ref, t_ref, o_ref):
    h = h_ref[...]
    r = r_ref[...]
    t = t_ref[...]
    sh = jnp.maximum(jnp.sum(jnp.abs(h), axis=1, keepdims=True), _EPS)
    st = jnp.maximum(jnp.sum(jnp.abs(t), axis=1, keepdims=True), _EPS)
    d = jnp.abs(h / sh + r - t / st)
    o_ref[...] = -jnp.sum(d, axis=1)


def _tc_score(hrows, rrows, trows):
    blk = 1024
    grid = BATCH // blk
    spec = pl.BlockSpec((blk, DIM), lambda i: (i, 0))
    return pl.pallas_call(
        _score_body,
        grid=(grid,),
        in_specs=[spec, spec, spec],
        out_specs=pl.BlockSpec((blk,), lambda i: (i,)),
        out_shape=jax.ShapeDtypeStruct((BATCH,), jnp.float32),
    )(hrows, rrows, trows)


def kernel(head_idxs, rel_idxs, tail_idxs, ent_emb, rel_emb):
    hidx = head_idxs.astype(jnp.int32)
    ridx = rel_idxs.astype(jnp.int32)
    tidx = tail_idxs.astype(jnp.int32)
    hrows, rrows, trows = _sc_gather(hidx, ridx, tidx, ent_emb, rel_emb)
    return _tc_score(hrows, rrows, trows)
